# Initial kernel scaffold; baseline (speedup 1.0000x reference)
import jax
import jax.numpy as jnp
from jax.experimental import pallas as pl


def kernel(x, edge_index, batch, W_emb, b_emb, W_rel, b_rel, W_root):
    raise NotImplementedError("write your pallas kernel here")


# speedup vs baseline: 3.4022x; 3.4022x over previous
import functools

import jax
import jax.numpy as jnp
from jax import lax
from jax.experimental import pallas as pl
from jax.experimental.pallas import tpu as pltpu
from jax.experimental.pallas import tpu_sc as plsc

N = 10000
E = 320000
D = 128
L = 3
G = 16

NC = 2
NS = 16
DH = D // NC
Np = 10240
EPT = E // NS
C = 80
ITERS = EPT // C
RPT = Np // NS
ZCH = 128
ZIT = RPT // ZCH

_mesh = plsc.VectorSubcoreMesh(
    core_axis_name="c", subcore_axis_name="s", num_cores=NC, num_subcores=NS
)


def _sc_agg_body(compute_deg, *refs):
    if compute_deg:
        (h2_hbm, src_hbm, dst_hbm, zrows_hbm, zdeg_hbm, ones_hbm,
         out_hbm, deg_hbm,
         sidx, sidx2, didx, rows, ones_v, stage, dstage, acc, dacc, sem) = refs
    else:
        (h2_hbm, src_hbm, dst_hbm, zrows_hbm,
         out_hbm,
         sidx, sidx2, didx, rows, stage, acc, sem) = refs

    cid = lax.axis_index("c")
    sid = lax.axis_index("s")

    pltpu.sync_copy(zrows_hbm, stage)
    if compute_deg:
        pltpu.sync_copy(zdeg_hbm, dstage)
    for k in range(ZIT):
        sl = pl.ds(sid * RPT + k * ZCH, ZCH)
        pltpu.sync_copy(stage, acc.at[sl])
        if compute_deg:
            pltpu.sync_copy(dstage, dacc.at[sl])
    if compute_deg:
        pltpu.sync_copy(ones_hbm, ones_v)
    plsc.subcore_barrier()

    base = sid * EPT

    def body(i, carry):
        off = base + i * C
        pltpu.sync_copy(src_hbm.at[pl.ds(off, C)], sidx)
        pltpu.sync_copy(dst_hbm.at[pl.ds(off, C)], didx)
        for j in range(C // 16):
            js = pl.ds(j * 16, 16)
            sidx2[js] = sidx[js] * 2 + cid
        pltpu.async_copy(h2_hbm.at[sidx2], rows, sem).wait()
        pltpu.sync_copy(rows, acc.at[didx], add=True)
        if compute_deg:
            @pl.when(cid == 0)
            def _():
                pltpu.sync_copy(ones_v, dacc.at[didx], add=True)
        return carry

    lax.fori_loop(0, ITERS, body, 0)
    plsc.subcore_barrier()

    for k in range(ZIT):
        sl = pl.ds(sid * RPT + k * ZCH, ZCH)
        pltpu.sync_copy(acc.at[sl], stage)
        pltpu.sync_copy(stage, out_hbm.at[cid].at[sl])
        if compute_deg:
            @pl.when(cid == 0)
            def _():
                pltpu.sync_copy(dacc.at[sl], dstage)
                pltpu.sync_copy(dstage, deg_hbm.at[sl])


_sc_agg_deg = functools.partial(
    pl.kernel,
    functools.partial(_sc_agg_body, True),
    out_type=(
        jax.ShapeDtypeStruct((NC, Np, DH), jnp.float32),
        jax.ShapeDtypeStruct((Np, 16), jnp.float32),
    ),
    mesh=_mesh,
    compiler_params=pltpu.CompilerParams(use_tc_tiling_on_sc=False),
    scratch_types=[
        pltpu.VMEM((C,), jnp.int32),
        pltpu.VMEM((C,), jnp.int32),
        pltpu.VMEM((C,), jnp.int32),
        pltpu.VMEM((C, DH), jnp.float32),
        pltpu.VMEM((C, 16), jnp.float32),
        pltpu.VMEM((ZCH, DH), jnp.float32),
        pltpu.VMEM((ZCH, 16), jnp.float32),
        pltpu.VMEM_SHARED((Np, DH), jnp.float32),
        pltpu.VMEM_SHARED((Np, 16), jnp.float32),
        pltpu.SemaphoreType.DMA,
    ],
)()

_sc_agg = functools.partial(
    pl.kernel,
    functools.partial(_sc_agg_body, False),
    out_type=jax.ShapeDtypeStruct((NC, Np, DH), jnp.float32),
    mesh=_mesh,
    compiler_params=pltpu.CompilerParams(use_tc_tiling_on_sc=False),
    scratch_types=[
        pltpu.VMEM((C,), jnp.int32),
        pltpu.VMEM((C,), jnp.int32),
        pltpu.VMEM((C,), jnp.int32),
        pltpu.VMEM((C, DH), jnp.float32),
        pltpu.VMEM((ZCH, DH), jnp.float32),
        pltpu.VMEM_SHARED((Np, DH), jnp.float32),
        pltpu.SemaphoreType.DMA,
    ],
)()


BN = 2048


def _emb_body(x_ref, w_ref, b_ref, o_ref):
    o_ref[...] = (
        jnp.dot(x_ref[...], w_ref[...], preferred_element_type=jnp.float32)
        + b_ref[...]
    )


def _tc_emb(x, W_emb, b_emb):
    return pl.pallas_call(
        _emb_body,
        grid=(Np // BN,),
        in_specs=[
            pl.BlockSpec((BN, D), lambda i: (i, 0)),
            pl.BlockSpec((D, D), lambda i: (0, 0)),
            pl.BlockSpec((1, D), lambda i: (0, 0)),
        ],
        out_specs=pl.BlockSpec((BN, D), lambda i: (i, 0)),
        out_shape=jax.ShapeDtypeStruct((Np, D), jnp.float32),
    )(x, W_emb, b_emb.reshape(1, D))


def _post_body(h_ref, p0_ref, p1_ref, d_ref, wr_ref, br_ref, wo_ref, o_ref):
    h = h_ref[...]
    deg = jnp.maximum(d_ref[:, :1], 1.0)
    agg = jnp.concatenate([p0_ref[0], p1_ref[0]], axis=1) / deg
    u = (
        jnp.dot(agg, wr_ref[...], preferred_element_type=jnp.float32)
        + br_ref[...]
        + jnp.dot(h, wo_ref[...], preferred_element_type=jnp.float32)
    )
    o_ref[...] = h + jnp.maximum(u, 0.0)


def _tc_post(h, parts, deg, W_rel_l, b_rel_l, W_root_l):
    return pl.pallas_call(
        _post_body,
        grid=(Np // BN,),
        in_specs=[
            pl.BlockSpec((BN, D), lambda i: (i, 0)),
            pl.BlockSpec((1, BN, DH), lambda i: (0, i, 0)),
            pl.BlockSpec((1, BN, DH), lambda i: (1, i, 0)),
            pl.BlockSpec((BN, 16), lambda i: (i, 0)),
            pl.BlockSpec((D, D), lambda i: (0, 0)),
            pl.BlockSpec((1, D), lambda i: (0, 0)),
            pl.BlockSpec((D, D), lambda i: (0, 0)),
        ],
        out_specs=pl.BlockSpec((BN, D), lambda i: (i, 0)),
        out_shape=jax.ShapeDtypeStruct((Np, D), jnp.float32),
    )(h, parts, parts, deg, W_rel_l, b_rel_l.reshape(1, D), W_root_l)


def _pool_body(h_ref, b_ref, o_ref):
    h = h_ref[...]
    bt = b_ref[...]
    gids = lax.broadcasted_iota(jnp.int32, (1, G), 1)
    mask = (bt == gids).astype(jnp.float32)
    cnt = jnp.sum(mask, axis=0, keepdims=True)
    maskn = mask / jnp.maximum(cnt, 1.0)
    o_ref[...] = lax.dot_general(
        maskn, h, (((0,), (0,)), ((), ())),
        preferred_element_type=jnp.float32,
    )


def _tc_pool(h, batch_pad):
    return pl.pallas_call(
        _pool_body,
        in_specs=[
            pl.BlockSpec((Np, D), lambda: (0, 0)),
            pl.BlockSpec((Np, 1), lambda: (0, 0)),
        ],
        out_specs=pl.BlockSpec((G, D), lambda: (0, 0)),
        out_shape=jax.ShapeDtypeStruct((G, D), jnp.float32),
    )(h, batch_pad.reshape(Np, 1))


def kernel(x, edge_index, batch, W_emb, b_emb, W_rel, b_rel, W_root):
    ei = edge_index.astype(jnp.int32)
    src = ei[0]
    dst = ei[1]
    x_pad = jnp.pad(x, ((0, Np - N), (0, 0)))
    batch_pad = jnp.pad(batch.astype(jnp.int32), (0, Np - N),
                        constant_values=G)

    zrows = jnp.zeros((ZCH, DH), jnp.float32)
    zdeg = jnp.zeros((ZCH, 16), jnp.float32)
    ones = jnp.ones((C, 16), jnp.float32)

    h = _tc_emb(x_pad, W_emb, b_emb)

    parts, deg = _sc_agg_deg(h.reshape(NC * Np, DH), src, dst,
                             zrows, zdeg, ones)
    h = _tc_post(h, parts, deg, W_rel[0], b_rel[0], W_root[0])

    for l in range(1, L):
        parts = _sc_agg(h.reshape(NC * Np, DH), src, dst, zrows)
        h = _tc_post(h, parts, deg, W_rel[l], b_rel[l], W_root[l])

    ge = _tc_pool(h, batch_pad)
    return h[:N], ge


# speedup vs baseline: 5.3949x; 1.5857x over previous
import functools

import jax
import jax.numpy as jnp
from jax import lax
from jax.experimental import pallas as pl
from jax.experimental.pallas import tpu as pltpu
from jax.experimental.pallas import tpu_sc as plsc

N = 10000
E = 320000
D = 128
L = 3
G = 16

NC = 2
NS = 16
DH = D // NC
Np = 10240
EPT = E // NS
C = 80
ITERS = EPT // C
CPT = ITERS
RPT = Np // NS
ZCH = 128
ZIT = RPT // ZCH

_mesh = plsc.VectorSubcoreMesh(
    core_axis_name="c", subcore_axis_name="s", num_cores=NC, num_subcores=NS
)


def _sc_agg_body(compute_deg, *refs):
    if compute_deg:
        (h2_hbm, src2_hbm, dst2_hbm, zrows_hbm, zdeg_hbm, ones_hbm,
         out_hbm, deg_hbm,
         sidxp, didxp, sidx2a, sidx2b, rowsa, rowsb, ones_v, stage, dstage,
         acc, dacc, gsa, gsb, ssa, ssb, dsa, dsb) = refs
    else:
        (h2_hbm, src2_hbm, dst2_hbm, zrows_hbm,
         out_hbm,
         sidxp, didxp, sidx2a, sidx2b, rowsa, rowsb, stage,
         acc, gsa, gsb, ssa, ssb) = refs

    cid = lax.axis_index("c")
    sid = lax.axis_index("s")

    pltpu.sync_copy(zrows_hbm, stage)
    if compute_deg:
        pltpu.sync_copy(zdeg_hbm, dstage)
    for k in range(ZIT):
        sl = pl.ds(sid * RPT + k * ZCH, ZCH)
        pltpu.sync_copy(stage, acc.at[sl])
        if compute_deg:
            pltpu.sync_copy(dstage, dacc.at[sl])
    if compute_deg:
        pltpu.sync_copy(ones_hbm, ones_v)
    plsc.subcore_barrier()

    base = sid * CPT

    def body(i, carry):
        row = base + 2 * i
        pltpu.sync_copy(src2_hbm.at[pl.ds(row, 2)], sidxp)
        pltpu.sync_copy(dst2_hbm.at[pl.ds(row, 2)], didxp)
        for j in range(C // 16):
            js = pl.ds(j * 16, 16)
            sidx2a[js] = sidxp[0, js] * 2 + cid
        ga = pltpu.async_copy(h2_hbm.at[sidx2a], rowsa, gsa)
        for j in range(C // 16):
            js = pl.ds(j * 16, 16)
            sidx2b[js] = sidxp[1, js] * 2 + cid
        gb = pltpu.async_copy(h2_hbm.at[sidx2b], rowsb, gsb)
        ga.wait()
        sa = pltpu.async_copy(rowsa, acc.at[didxp.at[0]], ssa, add=True)
        if compute_deg:
            @pl.when(cid == 0)
            def _():
                pltpu.async_copy(ones_v, dacc.at[didxp.at[0]], dsa,
                                 add=True).wait()
        gb.wait()
        sb = pltpu.async_copy(rowsb, acc.at[didxp.at[1]], ssb, add=True)
        if compute_deg:
            @pl.when(cid == 0)
            def _():
                pltpu.async_copy(ones_v, dacc.at[didxp.at[1]], dsb,
                                 add=True).wait()
        sa.wait()
        sb.wait()
        return carry

    lax.fori_loop(0, ITERS // 2, body, 0)
    plsc.subcore_barrier()

    for k in range(ZIT):
        sl = pl.ds(sid * RPT + k * ZCH, ZCH)
        pltpu.sync_copy(acc.at[sl], stage)
        pltpu.sync_copy(stage, out_hbm.at[cid].at[sl])
        if compute_deg:
            @pl.when(cid == 0)
            def _():
                pltpu.sync_copy(dacc.at[sl], dstage)
                pltpu.sync_copy(dstage, deg_hbm.at[sl])


_sc_agg_deg = functools.partial(
    pl.kernel,
    functools.partial(_sc_agg_body, True),
    out_type=(
        jax.ShapeDtypeStruct((NC, Np, DH), jnp.float32),
        jax.ShapeDtypeStruct((Np, 16), jnp.float32),
    ),
    mesh=_mesh,
    compiler_params=pltpu.CompilerParams(use_tc_tiling_on_sc=False),
    scratch_types=[
        pltpu.VMEM((2, C), jnp.int32),
        pltpu.VMEM((2, C), jnp.int32),
        pltpu.VMEM((C,), jnp.int32),
        pltpu.VMEM((C,), jnp.int32),
        pltpu.VMEM((C, DH), jnp.float32),
        pltpu.VMEM((C, DH), jnp.float32),
        pltpu.VMEM((C, 16), jnp.float32),
        pltpu.VMEM((ZCH, DH), jnp.float32),
        pltpu.VMEM((ZCH, 16), jnp.float32),
        pltpu.VMEM_SHARED((Np, DH), jnp.float32),
        pltpu.VMEM_SHARED((Np, 16), jnp.float32),
        pltpu.SemaphoreType.DMA,
        pltpu.SemaphoreType.DMA,
        pltpu.SemaphoreType.DMA,
        pltpu.SemaphoreType.DMA,
        pltpu.SemaphoreType.DMA,
        pltpu.SemaphoreType.DMA,
    ],
)()

_sc_agg = functools.partial(
    pl.kernel,
    functools.partial(_sc_agg_body, False),
    out_type=jax.ShapeDtypeStruct((NC, Np, DH), jnp.float32),
    mesh=_mesh,
    compiler_params=pltpu.CompilerParams(use_tc_tiling_on_sc=False),
    scratch_types=[
        pltpu.VMEM((2, C), jnp.int32),
        pltpu.VMEM((2, C), jnp.int32),
        pltpu.VMEM((C,), jnp.int32),
        pltpu.VMEM((C,), jnp.int32),
        pltpu.VMEM((C, DH), jnp.float32),
        pltpu.VMEM((C, DH), jnp.float32),
        pltpu.VMEM((ZCH, DH), jnp.float32),
        pltpu.VMEM_SHARED((Np, DH), jnp.float32),
        pltpu.SemaphoreType.DMA,
        pltpu.SemaphoreType.DMA,
        pltpu.SemaphoreType.DMA,
        pltpu.SemaphoreType.DMA,
    ],
)()


BN = 2048


def _emb_body(x_ref, w_ref, b_ref, o_ref):
    o_ref[...] = (
        jnp.dot(x_ref[...], w_ref[...], preferred_element_type=jnp.float32)
        + b_ref[...]
    )


def _tc_emb(x, W_emb, b_emb):
    return pl.pallas_call(
        _emb_body,
        grid=(Np // BN,),
        in_specs=[
            pl.BlockSpec((BN, D), lambda i: (i, 0)),
            pl.BlockSpec((D, D), lambda i: (0, 0)),
            pl.BlockSpec((1, D), lambda i: (0, 0)),
        ],
        out_specs=pl.BlockSpec((BN, D), lambda i: (i, 0)),
        out_shape=jax.ShapeDtypeStruct((Np, D), jnp.float32),
    )(x, W_emb, b_emb.reshape(1, D))


def _post_body(h_ref, p0_ref, p1_ref, d_ref, wr_ref, br_ref, wo_ref, o_ref):
    h = h_ref[...]
    deg = jnp.maximum(d_ref[:, :1], 1.0)
    agg = jnp.concatenate([p0_ref[0], p1_ref[0]], axis=1) / deg
    u = (
        jnp.dot(agg, wr_ref[...], preferred_element_type=jnp.float32)
        + br_ref[...]
        + jnp.dot(h, wo_ref[...], preferred_element_type=jnp.float32)
    )
    o_ref[...] = h + jnp.maximum(u, 0.0)


def _tc_post(h, parts, deg, W_rel_l, b_rel_l, W_root_l):
    return pl.pallas_call(
        _post_body,
        grid=(Np // BN,),
        in_specs=[
            pl.BlockSpec((BN, D), lambda i: (i, 0)),
            pl.BlockSpec((1, BN, DH), lambda i: (0, i, 0)),
            pl.BlockSpec((1, BN, DH), lambda i: (1, i, 0)),
            pl.BlockSpec((BN, 16), lambda i: (i, 0)),
            pl.BlockSpec((D, D), lambda i: (0, 0)),
            pl.BlockSpec((1, D), lambda i: (0, 0)),
            pl.BlockSpec((D, D), lambda i: (0, 0)),
        ],
        out_specs=pl.BlockSpec((BN, D), lambda i: (i, 0)),
        out_shape=jax.ShapeDtypeStruct((Np, D), jnp.float32),
    )(h, parts, parts, deg, W_rel_l, b_rel_l.reshape(1, D), W_root_l)


def _pool_body(h_ref, b_ref, o_ref):
    h = h_ref[...]
    bt = b_ref[...]
    gids = lax.broadcasted_iota(jnp.int32, (1, G), 1)
    mask = (bt == gids).astype(jnp.float32)
    cnt = jnp.sum(mask, axis=0, keepdims=True)
    maskn = mask / jnp.maximum(cnt, 1.0)
    o_ref[...] = lax.dot_general(
        maskn, h, (((0,), (0,)), ((), ())),
        preferred_element_type=jnp.float32,
    )


def _tc_pool(h, batch_pad):
    return pl.pallas_call(
        _pool_body,
        in_specs=[
            pl.BlockSpec((Np, D), lambda: (0, 0)),
            pl.BlockSpec((Np, 1), lambda: (0, 0)),
        ],
        out_specs=pl.BlockSpec((G, D), lambda: (0, 0)),
        out_shape=jax.ShapeDtypeStruct((G, D), jnp.float32),
    )(h, batch_pad.reshape(Np, 1))


def kernel(x, edge_index, batch, W_emb, b_emb, W_rel, b_rel, W_root):
    ei = edge_index.astype(jnp.int32)
    src = ei[0]
    dst = ei[1]
    x_pad = jnp.pad(x, ((0, Np - N), (0, 0)))
    batch_pad = jnp.pad(batch.astype(jnp.int32), (0, Np - N),
                        constant_values=G)

    zrows = jnp.zeros((ZCH, DH), jnp.float32)
    zdeg = jnp.zeros((ZCH, 16), jnp.float32)
    ones = jnp.ones((C, 16), jnp.float32)

    h = _tc_emb(x_pad, W_emb, b_emb)

    src2 = src.reshape(E // C, C)
    dst2 = dst.reshape(E // C, C)
    parts, deg = _sc_agg_deg(h.reshape(NC * Np, DH), src2, dst2,
                             zrows, zdeg, ones)
    h = _tc_post(h, parts, deg, W_rel[0], b_rel[0], W_root[0])

    for l in range(1, L):
        parts = _sc_agg(h.reshape(NC * Np, DH), src2, dst2, zrows)
        h = _tc_post(h, parts, deg, W_rel[l], b_rel[l], W_root[l])

    ge = _tc_pool(h, batch_pad)
    return h[:N], ge


# speedup vs baseline: 7.0754x; 1.3115x over previous
import functools

import jax
import jax.numpy as jnp
from jax import lax
from jax.experimental import pallas as pl
from jax.experimental.pallas import tpu as pltpu
from jax.experimental.pallas import tpu_sc as plsc

N = 10000
E = 320000
D = 128
L = 3
G = 16

NC = 2
NS = 16
DH = D // NC
Np = 10240
EPT = E // NS
C = 80
ITERS = EPT // C
CPT = ITERS
RPT = Np // NS
ZCH = 128
ZIT = RPT // ZCH

_mesh = plsc.VectorSubcoreMesh(
    core_axis_name="c", subcore_axis_name="s", num_cores=NC, num_subcores=NS
)


NB = 4
MAIN = ITERS // NB
TAIL = ITERS - MAIN * NB


def _sc_agg_body(compute_deg, *refs):
    nfix = 8 if compute_deg else 5
    if compute_deg:
        (h2_hbm, src2_hbm, dst2_hbm, zrows_hbm, zdeg_hbm, ones_hbm,
         out_hbm, deg_hbm) = refs[:nfix]
    else:
        (h2_hbm, src2_hbm, dst2_hbm, zrows_hbm, out_hbm) = refs[:nfix]
    r = list(refs[nfix:])
    sidxp = r.pop(0)
    didxp = r.pop(0)
    sidx2 = [r.pop(0) for _ in range(NB)]
    rows = [r.pop(0) for _ in range(NB)]
    gsem = [r.pop(0) for _ in range(NB)]
    ssem = [r.pop(0) for _ in range(NB)]
    if compute_deg:
        ones_v = r.pop(0)
        dstage = r.pop(0)
        dacc = r.pop(0)
        dsem = [r.pop(0) for _ in range(NB)]
    stage = r.pop(0)
    acc = r.pop(0)
    assert not r

    cid = lax.axis_index("c")
    sid = lax.axis_index("s")

    pltpu.sync_copy(zrows_hbm, stage)
    if compute_deg:
        pltpu.sync_copy(zdeg_hbm, dstage)
    for k in range(ZIT):
        sl = pl.ds(sid * RPT + k * ZCH, ZCH)
        pltpu.sync_copy(stage, acc.at[sl])
        if compute_deg:
            pltpu.sync_copy(dstage, dacc.at[sl])
    if compute_deg:
        pltpu.sync_copy(ones_hbm, ones_v)
    plsc.subcore_barrier()

    base = sid * CPT

    def run_chunks(row0, nb):
        pltpu.sync_copy(src2_hbm.at[pl.ds(row0, nb)], sidxp.at[pl.ds(0, nb)])
        pltpu.sync_copy(dst2_hbm.at[pl.ds(row0, nb)], didxp.at[pl.ds(0, nb)])
        gd = []
        for k in range(nb):
            for j in range(C // 16):
                js = pl.ds(j * 16, 16)
                sidx2[k][js] = sidxp[k, js] * 2 + cid
            gd.append(pltpu.async_copy(h2_hbm.at[sidx2[k]], rows[k], gsem[k]))
        sd = []
        for k in range(nb):
            gd[k].wait()
            sd.append(pltpu.async_copy(rows[k], acc.at[didxp.at[k]], ssem[k],
                                       add=True))
            if compute_deg:
                @pl.when(cid == 0)
                def _():
                    pltpu.async_copy(ones_v, dacc.at[didxp.at[k]], dsem[k],
                                     add=True).wait()
        for d in sd:
            d.wait()

    def body(i, carry):
        run_chunks(base + NB * i, NB)
        return carry

    lax.fori_loop(0, MAIN, body, 0)
    if TAIL:
        run_chunks(base + NB * MAIN, TAIL)
    plsc.subcore_barrier()

    for k in range(ZIT):
        sl = pl.ds(sid * RPT + k * ZCH, ZCH)
        pltpu.sync_copy(acc.at[sl], stage)
        pltpu.sync_copy(stage, out_hbm.at[cid].at[sl])
        if compute_deg:
            @pl.when(cid == 0)
            def _():
                pltpu.sync_copy(dacc.at[sl], dstage)
                pltpu.sync_copy(dstage, deg_hbm.at[sl])


def _mk_scratch(compute_deg):
    s = [
        pltpu.VMEM((NB, C), jnp.int32),
        pltpu.VMEM((NB, C), jnp.int32),
    ]
    s += [pltpu.VMEM((C,), jnp.int32) for _ in range(NB)]
    s += [pltpu.VMEM((C, DH), jnp.float32) for _ in range(NB)]
    s += [pltpu.SemaphoreType.DMA for _ in range(2 * NB)]
    if compute_deg:
        s += [
            pltpu.VMEM((C, 16), jnp.float32),
            pltpu.VMEM((ZCH, 16), jnp.float32),
            pltpu.VMEM_SHARED((Np, 16), jnp.float32),
        ]
        s += [pltpu.SemaphoreType.DMA for _ in range(NB)]
    s += [
        pltpu.VMEM((ZCH, DH), jnp.float32),
        pltpu.VMEM_SHARED((Np, DH), jnp.float32),
    ]
    return s


_sc_agg_deg = functools.partial(
    pl.kernel,
    functools.partial(_sc_agg_body, True),
    out_type=(
        jax.ShapeDtypeStruct((NC, Np, DH), jnp.float32),
        jax.ShapeDtypeStruct((Np, 16), jnp.float32),
    ),
    mesh=_mesh,
    compiler_params=pltpu.CompilerParams(use_tc_tiling_on_sc=False),
    scratch_types=_mk_scratch(True),
)()

_sc_agg = functools.partial(
    pl.kernel,
    functools.partial(_sc_agg_body, False),
    out_type=jax.ShapeDtypeStruct((NC, Np, DH), jnp.float32),
    mesh=_mesh,
    compiler_params=pltpu.CompilerParams(use_tc_tiling_on_sc=False),
    scratch_types=_mk_scratch(False),
)()


BN = 2048


def _emb_body(x_ref, w_ref, b_ref, o_ref):
    o_ref[...] = (
        jnp.dot(x_ref[...], w_ref[...], preferred_element_type=jnp.float32)
        + b_ref[...]
    )


def _tc_emb(x, W_emb, b_emb):
    return pl.pallas_call(
        _emb_body,
        grid=(Np // BN,),
        in_specs=[
            pl.BlockSpec((BN, D), lambda i: (i, 0)),
            pl.BlockSpec((D, D), lambda i: (0, 0)),
            pl.BlockSpec((1, D), lambda i: (0, 0)),
        ],
        out_specs=pl.BlockSpec((BN, D), lambda i: (i, 0)),
        out_shape=jax.ShapeDtypeStruct((Np, D), jnp.float32),
    )(x, W_emb, b_emb.reshape(1, D))


def _post_body(h_ref, p0_ref, p1_ref, d_ref, wr_ref, br_ref, wo_ref, o_ref):
    h = h_ref[...]
    deg = jnp.maximum(d_ref[:, :1], 1.0)
    agg = jnp.concatenate([p0_ref[0], p1_ref[0]], axis=1) / deg
    u = (
        jnp.dot(agg, wr_ref[...], preferred_element_type=jnp.float32)
        + br_ref[...]
        + jnp.dot(h, wo_ref[...], preferred_element_type=jnp.float32)
    )
    o_ref[...] = h + jnp.maximum(u, 0.0)


def _tc_post(h, parts, deg, W_rel_l, b_rel_l, W_root_l):
    return pl.pallas_call(
        _post_body,
        grid=(Np // BN,),
        in_specs=[
            pl.BlockSpec((BN, D), lambda i: (i, 0)),
            pl.BlockSpec((1, BN, DH), lambda i: (0, i, 0)),
            pl.BlockSpec((1, BN, DH), lambda i: (1, i, 0)),
            pl.BlockSpec((BN, 16), lambda i: (i, 0)),
            pl.BlockSpec((D, D), lambda i: (0, 0)),
            pl.BlockSpec((1, D), lambda i: (0, 0)),
            pl.BlockSpec((D, D), lambda i: (0, 0)),
        ],
        out_specs=pl.BlockSpec((BN, D), lambda i: (i, 0)),
        out_shape=jax.ShapeDtypeStruct((Np, D), jnp.float32),
    )(h, parts, parts, deg, W_rel_l, b_rel_l.reshape(1, D), W_root_l)


def _pool_body(h_ref, b_ref, o_ref):
    h = h_ref[...]
    bt = b_ref[...]
    gids = lax.broadcasted_iota(jnp.int32, (1, G), 1)
    mask = (bt == gids).astype(jnp.float32)
    cnt = jnp.sum(mask, axis=0, keepdims=True)
    maskn = mask / jnp.maximum(cnt, 1.0)
    o_ref[...] = lax.dot_general(
        maskn, h, (((0,), (0,)), ((), ())),
        preferred_element_type=jnp.float32,
    )


def _tc_pool(h, batch_pad):
    return pl.pallas_call(
        _pool_body,
        in_specs=[
            pl.BlockSpec((Np, D), lambda: (0, 0)),
            pl.BlockSpec((Np, 1), lambda: (0, 0)),
        ],
        out_specs=pl.BlockSpec((G, D), lambda: (0, 0)),
        out_shape=jax.ShapeDtypeStruct((G, D), jnp.float32),
    )(h, batch_pad.reshape(Np, 1))


def kernel(x, edge_index, batch, W_emb, b_emb, W_rel, b_rel, W_root):
    ei = edge_index.astype(jnp.int32)
    src = ei[0]
    dst = ei[1]
    x_pad = jnp.pad(x, ((0, Np - N), (0, 0)))
    batch_pad = jnp.pad(batch.astype(jnp.int32), (0, Np - N),
                        constant_values=G)

    zrows = jnp.zeros((ZCH, DH), jnp.float32)
    zdeg = jnp.zeros((ZCH, 16), jnp.float32)
    ones = jnp.ones((C, 16), jnp.float32)

    h = _tc_emb(x_pad, W_emb, b_emb)

    src2 = src.reshape(E // C, C)
    dst2 = dst.reshape(E // C, C)
    parts, deg = _sc_agg_deg(h.reshape(NC * Np, DH), src2, dst2,
                             zrows, zdeg, ones)
    h = _tc_post(h, parts, deg, W_rel[0], b_rel[0], W_root[0])

    for l in range(1, L):
        parts = _sc_agg(h.reshape(NC * Np, DH), src2, dst2, zrows)
        h = _tc_post(h, parts, deg, W_rel[l], b_rel[l], W_root[l])

    ge = _tc_pool(h, batch_pad)
    return h[:N], ge


# speedup vs baseline: 8.4821x; 1.1988x over previous
import functools

import jax
import jax.numpy as jnp
from jax import lax
from jax.experimental import pallas as pl
from jax.experimental.pallas import tpu as pltpu
from jax.experimental.pallas import tpu_sc as plsc

N = 10000
E = 320000
D = 128
L = 3
G = 16

NC = 2
NS = 16
DH = D // NC
Np = 10240
EPT = E // NS
C = 80
ITERS = EPT // C
CPT = ITERS
RPT = Np // NS
ZCH = 128
ZIT = RPT // ZCH

_mesh = plsc.VectorSubcoreMesh(
    core_axis_name="c", subcore_axis_name="s", num_cores=NC, num_subcores=NS
)


NB = 8
MAIN = ITERS // NB
TAIL = ITERS - MAIN * NB


def _sc_agg_body(compute_deg, *refs):
    nfix = 8 if compute_deg else 5
    if compute_deg:
        (h2_hbm, src2_hbm, dst2_hbm, zrows_hbm, zdeg_hbm, ones_hbm,
         out_hbm, deg_hbm) = refs[:nfix]
    else:
        (h2_hbm, src2_hbm, dst2_hbm, zrows_hbm, out_hbm) = refs[:nfix]
    r = list(refs[nfix:])
    sidxp = r.pop(0)
    didxp = r.pop(0)
    sidx2 = [r.pop(0) for _ in range(NB)]
    rows = [r.pop(0) for _ in range(NB)]
    gsem = [r.pop(0) for _ in range(NB)]
    ssem = [r.pop(0) for _ in range(NB)]
    if compute_deg:
        ones_v = r.pop(0)
        dstage = r.pop(0)
        dacc = r.pop(0)
        dsem = [r.pop(0) for _ in range(NB)]
    stage = r.pop(0)
    acc = r.pop(0)
    assert not r

    cid = lax.axis_index("c")
    sid = lax.axis_index("s")

    pltpu.sync_copy(zrows_hbm, stage)
    if compute_deg:
        pltpu.sync_copy(zdeg_hbm, dstage)
    for k in range(ZIT):
        sl = pl.ds(sid * RPT + k * ZCH, ZCH)
        pltpu.sync_copy(stage, acc.at[sl])
        if compute_deg:
            pltpu.sync_copy(dstage, dacc.at[sl])
    if compute_deg:
        pltpu.sync_copy(ones_hbm, ones_v)
    plsc.subcore_barrier()

    base = sid * CPT

    def run_chunks(row0, nb):
        pltpu.sync_copy(src2_hbm.at[pl.ds(row0, nb)], sidxp.at[pl.ds(0, nb)])
        pltpu.sync_copy(dst2_hbm.at[pl.ds(row0, nb)], didxp.at[pl.ds(0, nb)])
        gd = []
        for k in range(nb):
            for j in range(C // 16):
                js = pl.ds(j * 16, 16)
                sidx2[k][js] = sidxp[k, js] * 2 + cid
            gd.append(pltpu.async_copy(h2_hbm.at[sidx2[k]], rows[k], gsem[k]))
        sd = []
        for k in range(nb):
            gd[k].wait()
            sd.append(pltpu.async_copy(rows[k], acc.at[didxp.at[k]], ssem[k],
                                       add=True))
            if compute_deg:
                @pl.when(cid == 0)
                def _():
                    pltpu.async_copy(ones_v, dacc.at[didxp.at[k]], dsem[k],
                                     add=True).wait()
        for d in sd:
            d.wait()

    def body(i, carry):
        run_chunks(base + NB * i, NB)
        return carry

    lax.fori_loop(0, MAIN, body, 0)
    if TAIL:
        run_chunks(base + NB * MAIN, TAIL)
    plsc.subcore_barrier()

    for k in range(ZIT):
        sl = pl.ds(sid * RPT + k * ZCH, ZCH)
        pltpu.sync_copy(acc.at[sl], stage)
        pltpu.sync_copy(stage, out_hbm.at[cid].at[sl])
        if compute_deg:
            @pl.when(cid == 0)
            def _():
                pltpu.sync_copy(dacc.at[sl], dstage)
                pltpu.sync_copy(dstage, deg_hbm.at[sl])


def _mk_scratch(compute_deg):
    s = [
        pltpu.VMEM((NB, C), jnp.int32),
        pltpu.VMEM((NB, C), jnp.int32),
    ]
    s += [pltpu.VMEM((C,), jnp.int32) for _ in range(NB)]
    s += [pltpu.VMEM((C, DH), jnp.float32) for _ in range(NB)]
    s += [pltpu.SemaphoreType.DMA for _ in range(2 * NB)]
    if compute_deg:
        s += [
            pltpu.VMEM((C, 16), jnp.float32),
            pltpu.VMEM((ZCH, 16), jnp.float32),
            pltpu.VMEM_SHARED((Np, 16), jnp.float32),
        ]
        s += [pltpu.SemaphoreType.DMA for _ in range(NB)]
    s += [
        pltpu.VMEM((ZCH, DH), jnp.float32),
        pltpu.VMEM_SHARED((Np, DH), jnp.float32),
    ]
    return s


_sc_agg_deg = functools.partial(
    pl.kernel,
    functools.partial(_sc_agg_body, True),
    out_type=(
        jax.ShapeDtypeStruct((NC, Np, DH), jnp.float32),
        jax.ShapeDtypeStruct((Np, 16), jnp.float32),
    ),
    mesh=_mesh,
    compiler_params=pltpu.CompilerParams(use_tc_tiling_on_sc=False),
    scratch_types=_mk_scratch(True),
)()

_sc_agg = functools.partial(
    pl.kernel,
    functools.partial(_sc_agg_body, False),
    out_type=jax.ShapeDtypeStruct((NC, Np, DH), jnp.float32),
    mesh=_mesh,
    compiler_params=pltpu.CompilerParams(use_tc_tiling_on_sc=False),
    scratch_types=_mk_scratch(False),
)()


BN = 2048


def _emb_body(x_ref, w_ref, b_ref, o_ref):
    o_ref[...] = (
        jnp.dot(x_ref[...], w_ref[...], preferred_element_type=jnp.float32)
        + b_ref[...]
    )


def _tc_emb(x, W_emb, b_emb):
    return pl.pallas_call(
        _emb_body,
        grid=(Np // BN,),
        in_specs=[
            pl.BlockSpec((BN, D), lambda i: (i, 0)),
            pl.BlockSpec((D, D), lambda i: (0, 0)),
            pl.BlockSpec((1, D), lambda i: (0, 0)),
        ],
        out_specs=pl.BlockSpec((BN, D), lambda i: (i, 0)),
        out_shape=jax.ShapeDtypeStruct((Np, D), jnp.float32),
    )(x, W_emb, b_emb.reshape(1, D))


def _post_body(h_ref, p0_ref, p1_ref, d_ref, wr_ref, br_ref, wo_ref, o_ref):
    h = h_ref[...]
    deg = jnp.maximum(d_ref[:, :1], 1.0)
    agg = jnp.concatenate([p0_ref[0], p1_ref[0]], axis=1) / deg
    u = (
        jnp.dot(agg, wr_ref[...], preferred_element_type=jnp.float32)
        + br_ref[...]
        + jnp.dot(h, wo_ref[...], preferred_element_type=jnp.float32)
    )
    o_ref[...] = h + jnp.maximum(u, 0.0)


def _tc_post(h, parts, deg, W_rel_l, b_rel_l, W_root_l):
    return pl.pallas_call(
        _post_body,
        grid=(Np // BN,),
        in_specs=[
            pl.BlockSpec((BN, D), lambda i: (i, 0)),
            pl.BlockSpec((1, BN, DH), lambda i: (0, i, 0)),
            pl.BlockSpec((1, BN, DH), lambda i: (1, i, 0)),
            pl.BlockSpec((BN, 16), lambda i: (i, 0)),
            pl.BlockSpec((D, D), lambda i: (0, 0)),
            pl.BlockSpec((1, D), lambda i: (0, 0)),
            pl.BlockSpec((D, D), lambda i: (0, 0)),
        ],
        out_specs=pl.BlockSpec((BN, D), lambda i: (i, 0)),
        out_shape=jax.ShapeDtypeStruct((Np, D), jnp.float32),
    )(h, parts, parts, deg, W_rel_l, b_rel_l.reshape(1, D), W_root_l)


def _pool_body(h_ref, b_ref, o_ref):
    h = h_ref[...]
    bt = b_ref[...]
    gids = lax.broadcasted_iota(jnp.int32, (1, G), 1)
    mask = (bt == gids).astype(jnp.float32)
    cnt = jnp.sum(mask, axis=0, keepdims=True)
    maskn = mask / jnp.maximum(cnt, 1.0)
    o_ref[...] = lax.dot_general(
        maskn, h, (((0,), (0,)), ((), ())),
        preferred_element_type=jnp.float32,
    )


def _tc_pool(h, batch_pad):
    return pl.pallas_call(
        _pool_body,
        in_specs=[
            pl.BlockSpec((Np, D), lambda: (0, 0)),
            pl.BlockSpec((Np, 1), lambda: (0, 0)),
        ],
        out_specs=pl.BlockSpec((G, D), lambda: (0, 0)),
        out_shape=jax.ShapeDtypeStruct((G, D), jnp.float32),
    )(h, batch_pad.reshape(Np, 1))


def kernel(x, edge_index, batch, W_emb, b_emb, W_rel, b_rel, W_root):
    ei = edge_index.astype(jnp.int32)
    src = ei[0]
    dst = ei[1]
    x_pad = jnp.pad(x, ((0, Np - N), (0, 0)))
    batch_pad = jnp.pad(batch.astype(jnp.int32), (0, Np - N),
                        constant_values=G)

    zrows = jnp.zeros((ZCH, DH), jnp.float32)
    zdeg = jnp.zeros((ZCH, 16), jnp.float32)
    ones = jnp.ones((C, 16), jnp.float32)

    h = _tc_emb(x_pad, W_emb, b_emb)

    src2 = src.reshape(E // C, C)
    dst2 = dst.reshape(E // C, C)
    parts, deg = _sc_agg_deg(h.reshape(NC * Np, DH), src2, dst2,
                             zrows, zdeg, ones)
    h = _tc_post(h, parts, deg, W_rel[0], b_rel[0], W_root[0])

    for l in range(1, L):
        parts = _sc_agg(h.reshape(NC * Np, DH), src2, dst2, zrows)
        h = _tc_post(h, parts, deg, W_rel[l], b_rel[l], W_root[l])

    ge = _tc_pool(h, batch_pad)
    return h[:N], ge


# speedup vs baseline: 9.4911x; 1.1190x over previous
import functools

import jax
import jax.numpy as jnp
from jax import lax
from jax.experimental import pallas as pl
from jax.experimental.pallas import tpu as pltpu
from jax.experimental.pallas import tpu_sc as plsc

N = 10000
E = 320000
D = 128
L = 3
G = 16

NC = 2
NS = 16
DH = D // NC
Np = 10240
EPT = E // NS
C = 80
ITERS = EPT // C
CPT = ITERS
RPT = Np // NS
ZCH = 128
ZIT = RPT // ZCH

_mesh = plsc.VectorSubcoreMesh(
    core_axis_name="c", subcore_axis_name="s", num_cores=NC, num_subcores=NS
)


NB = 10
MAIN = ITERS // NB
TAIL = ITERS - MAIN * NB


def _sc_agg_body(compute_deg, *refs):
    nfix = 8 if compute_deg else 5
    if compute_deg:
        (h2_hbm, src2_hbm, dst2_hbm, zrows_hbm, zdeg_hbm, ones_hbm,
         out_hbm, deg_hbm) = refs[:nfix]
    else:
        (h2_hbm, src2_hbm, dst2_hbm, zrows_hbm, out_hbm) = refs[:nfix]
    r = list(refs[nfix:])
    sidxp = r.pop(0)
    didxp = r.pop(0)
    sidx2 = [r.pop(0) for _ in range(NB)]
    rows = [r.pop(0) for _ in range(NB)]
    gsem = [r.pop(0) for _ in range(NB)]
    ssem = [r.pop(0) for _ in range(NB)]
    if compute_deg:
        ones_v = r.pop(0)
        dstage = r.pop(0)
        dacc = r.pop(0)
        dsem = [r.pop(0) for _ in range(NB)]
    stage = r.pop(0)
    acc = r.pop(0)
    assert not r

    cid = lax.axis_index("c")
    sid = lax.axis_index("s")

    pltpu.sync_copy(zrows_hbm, stage)
    if compute_deg:
        pltpu.sync_copy(zdeg_hbm, dstage)
    for k in range(ZIT):
        sl = pl.ds(sid * RPT + k * ZCH, ZCH)
        pltpu.sync_copy(stage, acc.at[sl])
        if compute_deg:
            pltpu.sync_copy(dstage, dacc.at[sl])
    if compute_deg:
        pltpu.sync_copy(ones_hbm, ones_v)
    plsc.subcore_barrier()

    base = sid * CPT

    def run_chunks(row0, nb):
        pltpu.sync_copy(src2_hbm.at[pl.ds(row0, nb)], sidxp.at[pl.ds(0, nb)])
        pltpu.sync_copy(dst2_hbm.at[pl.ds(row0, nb)], didxp.at[pl.ds(0, nb)])
        gd = []
        for k in range(nb):
            for j in range(C // 16):
                js = pl.ds(j * 16, 16)
                sidx2[k][js] = sidxp[k, js] * 2 + cid
            gd.append(pltpu.async_copy(h2_hbm.at[sidx2[k]], rows[k], gsem[k]))
        sd = []
        for k in range(nb):
            gd[k].wait()
            sd.append(pltpu.async_copy(rows[k], acc.at[didxp.at[k]], ssem[k],
                                       add=True))
            if compute_deg:
                @pl.when(cid == 0)
                def _():
                    pltpu.async_copy(ones_v, dacc.at[didxp.at[k]], dsem[k],
                                     add=True).wait()
        for d in sd:
            d.wait()

    def body(i, carry):
        run_chunks(base + NB * i, NB)
        return carry

    lax.fori_loop(0, MAIN, body, 0)
    if TAIL:
        run_chunks(base + NB * MAIN, TAIL)
    plsc.subcore_barrier()

    for k in range(ZIT):
        sl = pl.ds(sid * RPT + k * ZCH, ZCH)
        pltpu.sync_copy(acc.at[sl], stage)
        pltpu.sync_copy(stage, out_hbm.at[cid].at[sl])
        if compute_deg:
            @pl.when(cid == 0)
            def _():
                pltpu.sync_copy(dacc.at[sl], dstage)
                pltpu.sync_copy(dstage, deg_hbm.at[sl])


def _mk_scratch(compute_deg):
    s = [
        pltpu.VMEM((NB, C), jnp.int32),
        pltpu.VMEM((NB, C), jnp.int32),
    ]
    s += [pltpu.VMEM((C,), jnp.int32) for _ in range(NB)]
    s += [pltpu.VMEM((C, DH), jnp.float32) for _ in range(NB)]
    s += [pltpu.SemaphoreType.DMA for _ in range(2 * NB)]
    if compute_deg:
        s += [
            pltpu.VMEM((C, 16), jnp.float32),
            pltpu.VMEM((ZCH, 16), jnp.float32),
            pltpu.VMEM_SHARED((Np, 16), jnp.float32),
        ]
        s += [pltpu.SemaphoreType.DMA for _ in range(NB)]
    s += [
        pltpu.VMEM((ZCH, DH), jnp.float32),
        pltpu.VMEM_SHARED((Np, DH), jnp.float32),
    ]
    return s


_sc_agg_deg = functools.partial(
    pl.kernel,
    functools.partial(_sc_agg_body, True),
    out_type=(
        jax.ShapeDtypeStruct((NC, Np, DH), jnp.float32),
        jax.ShapeDtypeStruct((Np, 16), jnp.float32),
    ),
    mesh=_mesh,
    compiler_params=pltpu.CompilerParams(use_tc_tiling_on_sc=False),
    scratch_types=_mk_scratch(True),
)()

_sc_agg = functools.partial(
    pl.kernel,
    functools.partial(_sc_agg_body, False),
    out_type=jax.ShapeDtypeStruct((NC, Np, DH), jnp.float32),
    mesh=_mesh,
    compiler_params=pltpu.CompilerParams(use_tc_tiling_on_sc=False),
    scratch_types=_mk_scratch(False),
)()


BN = 2048


def _emb_body(x_ref, w_ref, b_ref, o_ref):
    o_ref[...] = (
        jnp.dot(x_ref[...], w_ref[...], preferred_element_type=jnp.float32)
        + b_ref[...]
    )


def _tc_emb(x, W_emb, b_emb):
    return pl.pallas_call(
        _emb_body,
        grid=(Np // BN,),
        in_specs=[
            pl.BlockSpec((BN, D), lambda i: (i, 0)),
            pl.BlockSpec((D, D), lambda i: (0, 0)),
            pl.BlockSpec((1, D), lambda i: (0, 0)),
        ],
        out_specs=pl.BlockSpec((BN, D), lambda i: (i, 0)),
        out_shape=jax.ShapeDtypeStruct((Np, D), jnp.float32),
    )(x, W_emb, b_emb.reshape(1, D))


def _post_body(h_ref, p0_ref, p1_ref, d_ref, wr_ref, br_ref, wo_ref, o_ref):
    h = h_ref[...]
    deg = jnp.maximum(d_ref[:, :1], 1.0)
    agg = jnp.concatenate([p0_ref[0], p1_ref[0]], axis=1) / deg
    u = (
        jnp.dot(agg, wr_ref[...], preferred_element_type=jnp.float32)
        + br_ref[...]
        + jnp.dot(h, wo_ref[...], preferred_element_type=jnp.float32)
    )
    o_ref[...] = h + jnp.maximum(u, 0.0)


def _tc_post(h, parts, deg, W_rel_l, b_rel_l, W_root_l):
    return pl.pallas_call(
        _post_body,
        grid=(Np // BN,),
        in_specs=[
            pl.BlockSpec((BN, D), lambda i: (i, 0)),
            pl.BlockSpec((1, BN, DH), lambda i: (0, i, 0)),
            pl.BlockSpec((1, BN, DH), lambda i: (1, i, 0)),
            pl.BlockSpec((BN, 16), lambda i: (i, 0)),
            pl.BlockSpec((D, D), lambda i: (0, 0)),
            pl.BlockSpec((1, D), lambda i: (0, 0)),
            pl.BlockSpec((D, D), lambda i: (0, 0)),
        ],
        out_specs=pl.BlockSpec((BN, D), lambda i: (i, 0)),
        out_shape=jax.ShapeDtypeStruct((Np, D), jnp.float32),
    )(h, parts, parts, deg, W_rel_l, b_rel_l.reshape(1, D), W_root_l)


def _pool_body(h_ref, b_ref, o_ref):
    h = h_ref[...]
    bt = b_ref[...]
    gids = lax.broadcasted_iota(jnp.int32, (1, G), 1)
    mask = (bt == gids).astype(jnp.float32)
    cnt = jnp.sum(mask, axis=0, keepdims=True)
    maskn = mask / jnp.maximum(cnt, 1.0)
    o_ref[...] = lax.dot_general(
        maskn, h, (((0,), (0,)), ((), ())),
        preferred_element_type=jnp.float32,
    )


def _tc_pool(h, batch_pad):
    return pl.pallas_call(
        _pool_body,
        in_specs=[
            pl.BlockSpec((Np, D), lambda: (0, 0)),
            pl.BlockSpec((Np, 1), lambda: (0, 0)),
        ],
        out_specs=pl.BlockSpec((G, D), lambda: (0, 0)),
        out_shape=jax.ShapeDtypeStruct((G, D), jnp.float32),
    )(h, batch_pad.reshape(Np, 1))


def kernel(x, edge_index, batch, W_emb, b_emb, W_rel, b_rel, W_root):
    ei = edge_index.astype(jnp.int32)
    src = ei[0]
    dst = ei[1]
    x_pad = jnp.pad(x, ((0, Np - N), (0, 0)))
    batch_pad = jnp.pad(batch.astype(jnp.int32), (0, Np - N),
                        constant_values=G)

    zrows = jnp.zeros((ZCH, DH), jnp.float32)
    zdeg = jnp.zeros((ZCH, 16), jnp.float32)
    ones = jnp.ones((C, 16), jnp.float32)

    h = _tc_emb(x_pad, W_emb, b_emb)

    src2 = src.reshape(E // C, C)
    dst2 = dst.reshape(E // C, C)
    parts, deg = _sc_agg_deg(h.reshape(NC * Np, DH), src2, dst2,
                             zrows, zdeg, ones)
    h = _tc_post(h, parts, deg, W_rel[0], b_rel[0], W_root[0])

    for l in range(1, L):
        parts = _sc_agg(h.reshape(NC * Np, DH), src2, dst2, zrows)
        h = _tc_post(h, parts, deg, W_rel[l], b_rel[l], W_root[l])

    ge = _tc_pool(h, batch_pad)
    return h[:N], ge
